# baseline (speedup 1.0000x reference)
import functools

import jax
import jax.numpy as jnp
from jax import lax
from jax.experimental import pallas as pl
from jax.experimental.pallas import tpu as pltpu
from jax.experimental.pallas import tpu_sc as plsc

DICT_SIZE = 1000000
VEC = 32
BATCH = 16384
SUPER = 128
ROWS_PER_SUPER = SUPER // VEC
NSUPER = DICT_SIZE // ROWS_PER_SUPER

NUM_CORES = 2
NUM_SUBCORES = 16
LANES = 16
NW = NUM_CORES * NUM_SUBCORES
N_PER_W = BATCH // NW
CHUNK = 64
NCHUNK = N_PER_W // CHUNK
NGROUP = CHUNK // LANES


def _rsqrt_nr(s):
    i = plsc.bitcast(s, jnp.int32)
    y = plsc.bitcast(jnp.int32(0x5F3759DF) - (i >> 1), jnp.float32)
    for _ in range(3):
        y = y * (1.5 - 0.5 * s * y * y)
    return y


def _body(x1_hbm, x2_hbm, emb_hbm, out_hbm,
          idx1_v, idx2_v, sup1_v, sup2_v, off1_v, off2_v,
          r1a, r1b, r2a, r2b, out_v, sem0, sem1):
    wid = lax.axis_index("s") * NUM_CORES + lax.axis_index("c")

    pltpu.sync_copy(x1_hbm.at[wid], idx1_v)
    pltpu.sync_copy(x2_hbm.at[wid], idx2_v)

    def prep(t, carry):
        sl = pl.ds(t * LANES, LANES)
        v1 = idx1_v[sl]
        sup1_v[sl] = v1 >> 2
        off1_v[sl] = (v1 & 3) * VEC
        v2 = idx2_v[sl]
        sup2_v[sl] = v2 >> 2
        off2_v[sl] = (v2 & 3) * VEC
        return carry

    lax.fori_loop(0, N_PER_W // LANES, prep, 0)

    sems = [sem0, sem1]
    r1 = [r1a, r1b]
    r2 = [r2a, r2b]

    def fire(j):
        b = j % 2
        sl = pl.ds(j * CHUNK, CHUNK)
        return (pltpu.async_copy(emb_hbm.at[sup1_v.at[sl]], r1[b], sems[b]),
                pltpu.async_copy(emb_hbm.at[sup2_v.at[sl]], r2[b], sems[b]))

    lane = lax.iota(jnp.int32, LANES)
    inflight = fire(0)

    for j in range(NCHUNK):
        b = j % 2
        for c in inflight:
            c.wait()
        if j + 1 < NCHUNK:
            inflight = fire(j + 1)

        rows1 = r1[b]
        rows2 = r2[b]

        def step(g, carry):
            iv = g * LANES + lane
            o1 = off1_v[pl.ds(j * CHUNK + g * LANES, LANES)]
            o2 = off2_v[pl.ds(j * CHUNK + g * LANES, LANES)]
            dot = jnp.zeros((LANES,), jnp.float32)
            s1 = jnp.zeros((LANES,), jnp.float32)
            s2 = jnp.zeros((LANES,), jnp.float32)
            for d in range(VEC):
                e1 = plsc.load_gather(rows1, [iv, o1 + d])
                e2 = plsc.load_gather(rows2, [iv, o2 + d])
                dot = dot + e1 * e2
                s1 = s1 + e1 * e1
                s2 = s2 + e2 * e2
            scale1 = jnp.minimum(_rsqrt_nr(s1), 1.0)
            scale2 = jnp.minimum(_rsqrt_nr(s2), 1.0)
            out_v[pl.ds(j * CHUNK + g * LANES, LANES)] = dot * scale1 * scale2
            return carry

        lax.fori_loop(0, NGROUP, step, 0)

    pltpu.sync_copy(out_v, out_hbm.at[pl.ds(wid * N_PER_W, N_PER_W)])


@jax.jit
def _run(x1, x2, embedding):
    mesh = plsc.VectorSubcoreMesh(
        core_axis_name="c", subcore_axis_name="s",
        num_cores=NUM_CORES, num_subcores=NUM_SUBCORES)
    f = pl.kernel(
        _body,
        out_type=jax.ShapeDtypeStruct((BATCH,), jnp.float32),
        mesh=mesh,
        scratch_types=[
            pltpu.VMEM((N_PER_W,), jnp.int32),
            pltpu.VMEM((N_PER_W,), jnp.int32),
            pltpu.VMEM((N_PER_W,), jnp.int32),
            pltpu.VMEM((N_PER_W,), jnp.int32),
            pltpu.VMEM((N_PER_W,), jnp.int32),
            pltpu.VMEM((N_PER_W,), jnp.int32),
            pltpu.VMEM((CHUNK, SUPER), jnp.float32),
            pltpu.VMEM((CHUNK, SUPER), jnp.float32),
            pltpu.VMEM((CHUNK, SUPER), jnp.float32),
            pltpu.VMEM((CHUNK, SUPER), jnp.float32),
            pltpu.VMEM((N_PER_W,), jnp.float32),
            pltpu.SemaphoreType.DMA,
            pltpu.SemaphoreType.DMA,
        ],
        compiler_params=pltpu.CompilerParams(needs_layout_passes=False),
    )
    x1r = x1.reshape(NW, N_PER_W)
    x2r = x2.reshape(NW, N_PER_W)
    embr = embedding.reshape(NSUPER, SUPER)
    return f(x1r, x2r, embr)


def kernel(x1, x2, embedding):
    return _run(x1, x2, embedding)


# speedup vs baseline: 1.0071x; 1.0071x over previous
import functools

import jax
import jax.numpy as jnp
from jax import lax
from jax.experimental import pallas as pl
from jax.experimental.pallas import tpu as pltpu
from jax.experimental.pallas import tpu_sc as plsc

DICT_SIZE = 1000000
VEC = 32
BATCH = 16384

NUM_CORES = 2
NUM_SUBCORES = 16
LANES = 16
NW = NUM_CORES * NUM_SUBCORES
N_PER_W = BATCH // NW
CHUNK = 64
NCHUNK = N_PER_W // CHUNK
NGROUP = CHUNK // LANES


def _rsqrt_nr(s):
    i = plsc.bitcast(s, jnp.int32)
    y = plsc.bitcast(jnp.int32(0x5F3759DF) - (i >> 1), jnp.float32)
    for _ in range(3):
        y = y * (1.5 - 0.5 * s * y * y)
    return y


def _body(x1_hbm, x2_hbm, emb_hbm, out_hbm,
          idx1_v, idx2_v, sup1_v, sup2_v, off1_v, off2_v,
          r1a, r1b, r2a, r2b, out_v, sem0, sem1):
    wid = lax.axis_index("s") * NUM_CORES + lax.axis_index("c")

    pltpu.sync_copy(x1_hbm.at[wid], idx1_v)
    pltpu.sync_copy(x2_hbm.at[wid], idx2_v)

    def prep(t, carry):
        sl = pl.ds(t * LANES, LANES)
        v1 = idx1_v[sl]
        sup1_v[sl] = v1
        off1_v[sl] = v1
        v2 = idx2_v[sl]
        sup2_v[sl] = v2
        off2_v[sl] = v2
        return carry

    lax.fori_loop(0, N_PER_W // LANES, prep, 0)

    sems = [sem0, sem1]
    r1 = [r1a, r1b]
    r2 = [r2a, r2b]

    def fire(j):
        b = j % 2
        sl = pl.ds(j * CHUNK, CHUNK)
        return (pltpu.async_copy(emb_hbm.at[sup1_v.at[sl]], r1[b], sems[b]),
                pltpu.async_copy(emb_hbm.at[sup2_v.at[sl]], r2[b], sems[b]))

    lane = lax.iota(jnp.int32, LANES)
    inflight = fire(0)

    for j in range(NCHUNK):
        b = j % 2
        for c in inflight:
            c.wait()
        if j + 1 < NCHUNK:
            inflight = fire(j + 1)

        rows1 = r1[b]
        rows2 = r2[b]

        def step(g, carry):
            iv = g * LANES + lane
            o1 = off1_v[pl.ds(j * CHUNK + g * LANES, LANES)]
            o2 = off2_v[pl.ds(j * CHUNK + g * LANES, LANES)]
            dot = jnp.zeros((LANES,), jnp.float32)
            s1 = jnp.zeros((LANES,), jnp.float32)
            s2 = jnp.zeros((LANES,), jnp.float32)
            for d in range(VEC):
                dv = jnp.full((LANES,), d, jnp.int32)
                e1 = plsc.load_gather(rows1, [iv, dv])
                e2 = plsc.load_gather(rows2, [iv, dv])
                dot = dot + e1 * e2
                s1 = s1 + e1 * e1
                s2 = s2 + e2 * e2
            scale1 = jnp.minimum(_rsqrt_nr(s1), 1.0)
            scale2 = jnp.minimum(_rsqrt_nr(s2), 1.0)
            out_v[pl.ds(j * CHUNK + g * LANES, LANES)] = dot * scale1 * scale2
            return carry

        lax.fori_loop(0, NGROUP, step, 0)

    pltpu.sync_copy(out_v, out_hbm.at[pl.ds(wid * N_PER_W, N_PER_W)])


@jax.jit
def _run(x1, x2, embedding):
    mesh = plsc.VectorSubcoreMesh(
        core_axis_name="c", subcore_axis_name="s",
        num_cores=NUM_CORES, num_subcores=NUM_SUBCORES)
    f = pl.kernel(
        _body,
        out_type=jax.ShapeDtypeStruct((BATCH,), jnp.float32),
        mesh=mesh,
        scratch_types=[
            pltpu.VMEM((N_PER_W,), jnp.int32),
            pltpu.VMEM((N_PER_W,), jnp.int32),
            pltpu.VMEM((N_PER_W,), jnp.int32),
            pltpu.VMEM((N_PER_W,), jnp.int32),
            pltpu.VMEM((N_PER_W,), jnp.int32),
            pltpu.VMEM((N_PER_W,), jnp.int32),
            pltpu.VMEM((CHUNK, VEC), jnp.float32),
            pltpu.VMEM((CHUNK, VEC), jnp.float32),
            pltpu.VMEM((CHUNK, VEC), jnp.float32),
            pltpu.VMEM((CHUNK, VEC), jnp.float32),
            pltpu.VMEM((N_PER_W,), jnp.float32),
            pltpu.SemaphoreType.DMA,
            pltpu.SemaphoreType.DMA,
        ],
        compiler_params=pltpu.CompilerParams(needs_layout_passes=False, use_tc_tiling_on_sc=False),
    )
    x1r = x1.reshape(NW, N_PER_W)
    x2r = x2.reshape(NW, N_PER_W)
    return f(x1r, x2r, embedding)


def kernel(x1, x2, embedding):
    return _run(x1, x2, embedding)
